# baseline (speedup 1.0000x reference)
import functools

import jax
import jax.numpy as jnp
from jax import lax
from jax.experimental import pallas as pl
from jax.experimental.pallas import tpu as pltpu
from jax.experimental.pallas import tpu_sc as plsc

N_NODES = 10000
N_PAD = 10240
E_EDGES = 320000
E_PAD = 327680
P_EDGES = 65536
D_FEAT = 128
R_DIM = 32
NUM_RELS = 200

NC = 2
NS = 16
NW = NC * NS
ECHUNK = 64
EC = 160
DCHUNK = 128
DC = 80
PC = 16
ROWS_PER_SUB = N_PAD // NS

_mesh = plsc.VectorSubcoreMesh(core_axis_name="c", subcore_axis_name="s")


@functools.partial(
    pl.kernel,
    out_type=jax.ShapeDtypeStruct((NC, 2, N_PAD, 8), jnp.float32),
    mesh=_mesh,
    scratch_types=[
        pltpu.VMEM((DC, 128), jnp.int32),
        pltpu.VMEM((DC, 128), jnp.int32),
        pltpu.VMEM((128, 8), jnp.float32),
        pltpu.SemaphoreType.DMA,
        pltpu.SemaphoreType.DMA,
        pltpu.VMEM_SHARED((N_PAD, 8), jnp.float32),
        pltpu.VMEM_SHARED((N_PAD, 8), jnp.float32),
    ],
)
def _sc_degrees(src_hbm, dst_hbm, zeros_hbm, ones_hbm, out_hbm,
                src_v, dst_v, ones_v, sem_s, sem_d, dsrc_sh, ddst_sh):
    c = lax.axis_index("c")
    s = lax.axis_index("s")
    wid = c * NS + s
    base = s * ROWS_PER_SUB
    pltpu.sync_copy(zeros_hbm, dsrc_sh.at[pl.ds(base, ROWS_PER_SUB)])
    pltpu.sync_copy(zeros_hbm, ddst_sh.at[pl.ds(base, ROWS_PER_SUB)])
    pltpu.sync_copy(ones_hbm, ones_v)
    pltpu.sync_copy(src_hbm.at[wid], src_v)
    pltpu.sync_copy(dst_hbm.at[wid], dst_v)
    plsc.subcore_barrier()

    pltpu.async_copy(ones_v, dsrc_sh.at[src_v.at[0]], sem_s, add=True)
    pltpu.async_copy(ones_v, ddst_sh.at[dst_v.at[0]], sem_d, add=True)

    def body(j, carry):
        pltpu.async_copy(ones_v, dsrc_sh.at[src_v.at[j + 1]], sem_s, add=True)
        pltpu.async_copy(ones_v, ddst_sh.at[dst_v.at[j + 1]], sem_d, add=True)
        pltpu.make_async_copy(ones_v, dsrc_sh.at[src_v.at[j]], sem_s).wait()
        pltpu.make_async_copy(ones_v, ddst_sh.at[dst_v.at[j]], sem_d).wait()
        return carry

    lax.fori_loop(0, DC - 1, body, 0)
    pltpu.make_async_copy(ones_v, dsrc_sh.at[src_v.at[0]], sem_s).wait()
    pltpu.make_async_copy(ones_v, ddst_sh.at[dst_v.at[0]], sem_d).wait()
    plsc.subcore_barrier()
    pltpu.sync_copy(dsrc_sh.at[pl.ds(base, ROWS_PER_SUB)],
                    out_hbm.at[c, 0, pl.ds(base, ROWS_PER_SUB)])
    pltpu.sync_copy(ddst_sh.at[pl.ds(base, ROWS_PER_SUB)],
                    out_hbm.at[c, 1, pl.ds(base, ROWS_PER_SUB)])


NBUF = 5
GRP = EC // NBUF
HALF = D_FEAT // 2


@functools.partial(
    pl.kernel,
    out_type=jax.ShapeDtypeStruct((NC, 2, N_PAD, HALF), jnp.float32),
    mesh=_mesh,
    scratch_types=[
        pltpu.VMEM((EC, ECHUNK), jnp.int32),
        pltpu.VMEM((EC, ECHUNK), jnp.int32),
        pltpu.VMEM((ECHUNK, HALF), jnp.float32),
        pltpu.VMEM((ECHUNK, HALF), jnp.float32),
        pltpu.VMEM((ECHUNK, HALF), jnp.float32),
        pltpu.VMEM((ECHUNK, HALF), jnp.float32),
        pltpu.VMEM((ECHUNK, HALF), jnp.float32),
        pltpu.SemaphoreType.DMA,
        pltpu.SemaphoreType.DMA,
        pltpu.SemaphoreType.DMA,
        pltpu.SemaphoreType.DMA,
        pltpu.SemaphoreType.DMA,
        pltpu.VMEM_SHARED((N_PAD, HALF), jnp.float32),
        pltpu.VMEM_SHARED((N_PAD, HALF), jnp.float32),
    ],
    compiler_params=pltpu.CompilerParams(use_tc_tiling_on_sc=False),
)
def _sc_aggregate(h_hbm, src_hbm, dst_hbm, zeros_hbm, out_hbm,
                  src_v, dst_v, r0, r1, r2, r3, r4,
                  sem0, sem1, sem2, sem3, sem4,
                  acc_sh, h_sh):
    rows = (r0, r1, r2, r3, r4)
    sems = (sem0, sem1, sem2, sem3, sem4)
    c = lax.axis_index("c")
    s = lax.axis_index("s")
    wid = c * NS + s
    base = s * ROWS_PER_SUB
    pltpu.async_copy(src_hbm.at[wid], src_v, sem0).wait()
    pltpu.async_copy(dst_hbm.at[wid], dst_v, sem1).wait()

    def half_body(q, carry):
        pltpu.async_copy(h_hbm.at[q].at[pl.ds(base, ROWS_PER_SUB)],
                         h_sh.at[pl.ds(base, ROWS_PER_SUB)], sem2).wait()
        pltpu.async_copy(zeros_hbm, acc_sh.at[pl.ds(base, ROWS_PER_SUB)],
                         sem0).wait()
        plsc.subcore_barrier()
        for b in range(NBUF):
            pltpu.async_copy(h_sh.at[src_v.at[b]], rows[b], sems[b])

        def group(g, carry2):
            for b in range(NBUF):
                j = g * NBUF + b
                pltpu.make_async_copy(
                    h_sh.at[src_v.at[j]], rows[b], sems[b]).wait()
                pltpu.async_copy(rows[b], acc_sh.at[dst_v.at[j]], sems[b],
                                 add=True)

            @pl.when(g < GRP - 1)
            def _():
                for b in range(NBUF):
                    j = g * NBUF + b
                    pltpu.make_async_copy(
                        rows[b], acc_sh.at[dst_v.at[j]], sems[b]).wait()
                    pltpu.async_copy(
                        h_sh.at[src_v.at[j + NBUF]], rows[b], sems[b])

            @pl.when(g == GRP - 1)
            def _():
                for b in range(NBUF):
                    j = g * NBUF + b
                    pltpu.make_async_copy(
                        rows[b], acc_sh.at[dst_v.at[j]], sems[b]).wait()
            return carry2

        lax.fori_loop(0, GRP, group, 0)
        plsc.subcore_barrier()
        pltpu.async_copy(acc_sh.at[pl.ds(base, ROWS_PER_SUB)],
                         out_hbm.at[c, q, pl.ds(base, ROWS_PER_SUB)],
                         sem0).wait()
        plsc.subcore_barrier()
        return carry

    lax.fori_loop(0, 2, half_body, 0)


@functools.partial(
    pl.kernel,
    out_type=[jax.ShapeDtypeStruct((P_EDGES,), jnp.float32)
              for _ in range(2)],
    mesh=_mesh,
    scratch_types=[
        pltpu.VMEM((PC, 128), jnp.int32),
        pltpu.VMEM((PC, 128), jnp.int32),
        pltpu.VMEM((PC, 128), jnp.int32),
        pltpu.VMEM((PC, 128), jnp.int32),
        pltpu.VMEM((128, R_DIM), jnp.float32),
        pltpu.VMEM((128, R_DIM), jnp.float32),
        pltpu.VMEM((128, R_DIM), jnp.float32),
        pltpu.VMEM((128, R_DIM), jnp.float32),
        pltpu.VMEM((128,), jnp.float32),
        pltpu.VMEM((128,), jnp.float32),
        pltpu.SemaphoreType.DMA,
        pltpu.SemaphoreType.DMA,
        pltpu.SemaphoreType.DMA,
        pltpu.SemaphoreType.DMA,
        pltpu.VMEM_SHARED((N_PAD, R_DIM), jnp.float32),
        pltpu.VMEM_SHARED((NUM_RELS, R_DIM), jnp.float32),
    ],
    compiler_params=pltpu.CompilerParams(use_tc_tiling_on_sc=False,
                                         needs_layout_passes=False),
)
def _sc_scores(xn_hbm, rel_hbm, psrc_hbm, pdst_hbm, ndst_hbm, ptype_hbm,
               pos_hbm, neg_hbm,
               ihe, ite, itn, ire, he_v, te_v, tn_v, re_v, pos_c, neg_c,
               sem0, sem1, sem2, sem3, xn_sh, rel_sh):
    c = lax.axis_index("c")
    s = lax.axis_index("s")
    wid = c * NS + s
    base = s * ROWS_PER_SUB
    obase = wid * (PC * 128)
    pltpu.async_copy(xn_hbm.at[pl.ds(base, ROWS_PER_SUB)],
                     xn_sh.at[pl.ds(base, ROWS_PER_SUB)], sem0).wait()

    @pl.when(s == 0)
    def _():
        pltpu.async_copy(rel_hbm, rel_sh, sem1).wait()

    pltpu.async_copy(psrc_hbm.at[wid], ihe, sem0).wait()
    pltpu.async_copy(pdst_hbm.at[wid], ite, sem1).wait()
    pltpu.async_copy(ndst_hbm.at[wid], itn, sem2).wait()
    pltpu.async_copy(ptype_hbm.at[wid], ire, sem3).wait()
    plsc.subcore_barrier()

    def body(j, carry):
        pltpu.async_copy(xn_sh.at[ihe.at[j]], he_v, sem0)
        pltpu.async_copy(xn_sh.at[ite.at[j]], te_v, sem1)
        pltpu.async_copy(xn_sh.at[itn.at[j]], tn_v, sem2)
        pltpu.async_copy(rel_sh.at[ire.at[j]], re_v, sem3)
        pltpu.make_async_copy(xn_sh.at[ihe.at[j]], he_v, sem0).wait()
        pltpu.make_async_copy(xn_sh.at[ite.at[j]], te_v, sem1).wait()
        pltpu.make_async_copy(xn_sh.at[itn.at[j]], tn_v, sem2).wait()
        pltpu.make_async_copy(rel_sh.at[ire.at[j]], re_v, sem3).wait()
        lanes = lax.iota(jnp.int32, 16)
        for e16 in range(8):
            pv = jnp.zeros((16,), jnp.float32)
            nv = jnp.zeros((16,), jnp.float32)
            for l in range(16):
                e = e16 * 16 + l
                hr0 = he_v[e, pl.ds(0, 16)] + re_v[e, pl.ds(0, 16)]
                hr1 = he_v[e, pl.ds(16, 16)] + re_v[e, pl.ds(16, 16)]
                dp0 = hr0 - te_v[e, pl.ds(0, 16)]
                dp1 = hr1 - te_v[e, pl.ds(16, 16)]
                dn0 = hr0 - tn_v[e, pl.ds(0, 16)]
                dn1 = hr1 - tn_v[e, pl.ds(16, 16)]
                pv = jnp.where(lanes == l, jnp.sum(dp0 * dp0 + dp1 * dp1), pv)
                nv = jnp.where(lanes == l, jnp.sum(dn0 * dn0 + dn1 * dn1), nv)
            pos_c[pl.ds(e16 * 16, 16)] = pv
            neg_c[pl.ds(e16 * 16, 16)] = nv
        pltpu.async_copy(pos_c, pos_hbm.at[pl.ds(obase + j * 128, 128)],
                         sem0).wait()
        pltpu.async_copy(neg_c, neg_hbm.at[pl.ds(obase + j * 128, 128)],
                         sem1).wait()
        return carry

    lax.fori_loop(0, PC, body, 0)


def _tc_norms_body(dp_ref, feat_ref, h1_ref, nsrc_ref, ndst_ref):
    dsrc = dp_ref[0, 0] + dp_ref[1, 0]
    ddst = dp_ref[0, 1] + dp_ref[1, 1]
    dsrc1 = dsrc[:, 0:1]
    ddst1 = ddst[:, 0:1]
    nsrc = jnp.where(dsrc1 > 0.0, lax.rsqrt(dsrc1), 0.0)
    ndst = jnp.where(ddst1 > 0.0, lax.rsqrt(ddst1), 0.0)
    nsrc_ref[...] = jnp.broadcast_to(nsrc, (N_PAD, D_FEAT))
    ndst_ref[...] = jnp.broadcast_to(ndst, (N_PAD, D_FEAT))
    h1 = feat_ref[...] * nsrc
    h1_ref[0] = h1[:, :HALF]
    h1_ref[1] = h1[:, HALF:]


def _sum_parts(p_ref, ndst_ref):
    lo = p_ref[0, 0] + p_ref[1, 0]
    hi = p_ref[0, 1] + p_ref[1, 1]
    return jnp.concatenate([lo, hi], axis=1) * ndst_ref[...]


def _tc_layer1_body(p_ref, ndst_ref, nsrc_ref, w_ref, b_ref, h2_ref):
    agg = _sum_parts(p_ref, ndst_ref)
    x1 = jnp.tanh(jnp.dot(agg, w_ref[...],
                          preferred_element_type=jnp.float32) + b_ref[...])
    h2 = x1 * nsrc_ref[...]
    h2_ref[0] = h2[:, :HALF]
    h2_ref[1] = h2[:, HALF:]


def _tc_layer2_body(p_ref, ndst_ref, feat_ref, w_ref, b_ref, wl_ref, rel_ref,
                    xn_ref, reln_ref):
    agg = _sum_parts(p_ref, ndst_ref)
    x2 = jnp.tanh(jnp.dot(agg, w_ref[...],
                          preferred_element_type=jnp.float32) + b_ref[...])
    x = (jnp.dot(x2, wl_ref[0:D_FEAT], preferred_element_type=jnp.float32)
         + jnp.dot(feat_ref[...], wl_ref[D_FEAT:],
                   preferred_element_type=jnp.float32))
    n = jnp.sqrt(jnp.sum(x * x, axis=1, keepdims=True))
    xn_ref[...] = x / jnp.maximum(n, 1e-12)
    r = rel_ref[...]
    rn = jnp.sqrt(jnp.sum(r * r, axis=1, keepdims=True))
    reln_ref[...] = r / jnp.maximum(rn, 1.0)


def _tc_sqrt_body(p_ref, n_ref, pos_ref, neg_ref):
    pos_ref[...] = jnp.sqrt(p_ref[...])
    neg_ref[...] = jnp.sqrt(n_ref[...])


def kernel(input_feat, edge_src, edge_dst, pos_src, pos_dst, pos_type, neg_dst,
           W1, b1, W2, b2, W_lin, rel_table):
    f32 = jnp.float32
    feat_p = jnp.pad(input_feat, ((0, N_PAD - N_NODES), (0, 0)))
    epad = E_PAD - E_EDGES
    src_flat = jnp.concatenate(
        [edge_src, jnp.full((epad,), N_NODES, jnp.int32)])
    dst_flat = jnp.concatenate(
        [edge_dst, jnp.full((epad,), N_NODES, jnp.int32)])
    src_d = src_flat.reshape(NW, DC, DCHUNK)
    dst_d = dst_flat.reshape(NW, DC, DCHUNK)
    src_p = src_flat.reshape(NW, EC, ECHUNK)
    dst_p = dst_flat.reshape(NW, EC, ECHUNK)
    psrc_r = pos_src.reshape(NW, PC, 128)
    pdst_r = pos_dst.reshape(NW, PC, 128)
    ndst_r = neg_dst.reshape(NW, PC, 128)
    ptype_r = pos_type.reshape(NW, PC, 128)
    zeros_blk = jnp.zeros((ROWS_PER_SUB, HALF), f32)
    zeros_blk8 = jnp.zeros((ROWS_PER_SUB, 8), f32)
    ones_blk = jnp.ones((128, 8), f32)

    deg_part = _sc_degrees(src_d, dst_d, zeros_blk8, ones_blk)
    h1, nsrc_b, ndst_b = pl.pallas_call(
        _tc_norms_body,
        out_shape=[jax.ShapeDtypeStruct((2, N_PAD, HALF), f32),
                   jax.ShapeDtypeStruct((N_PAD, D_FEAT), f32),
                   jax.ShapeDtypeStruct((N_PAD, D_FEAT), f32)],
    )(deg_part, feat_p)

    p1 = _sc_aggregate(h1, src_p, dst_p, zeros_blk)
    h2 = pl.pallas_call(
        _tc_layer1_body,
        out_shape=jax.ShapeDtypeStruct((2, N_PAD, HALF), f32),
    )(p1, ndst_b, nsrc_b, W1, b1)

    p2 = _sc_aggregate(h2, src_p, dst_p, zeros_blk)
    xn, rel_n = pl.pallas_call(
        _tc_layer2_body,
        out_shape=[jax.ShapeDtypeStruct((N_PAD, R_DIM), f32),
                   jax.ShapeDtypeStruct((NUM_RELS, R_DIM), f32)],
    )(p2, ndst_b, feat_p, W2, b2, W_lin, rel_table)

    pos_sq, neg_sq = _sc_scores(xn, rel_n, psrc_r, pdst_r, ndst_r, ptype_r)
    pos, neg = pl.pallas_call(
        _tc_sqrt_body,
        out_shape=[jax.ShapeDtypeStruct((P_EDGES,), f32)] * 2,
    )(pos_sq, neg_sq)
    return (pos, neg)


# speedup vs baseline: 1.1193x; 1.1193x over previous
import functools

import jax
import jax.numpy as jnp
from jax import lax
from jax.experimental import pallas as pl
from jax.experimental.pallas import tpu as pltpu
from jax.experimental.pallas import tpu_sc as plsc

N_NODES = 10000
N_PAD = 10240
E_EDGES = 320000
E_PAD = 327680
P_EDGES = 65536
D_FEAT = 128
R_DIM = 32
NUM_RELS = 200

NC = 2
NS = 16
NW = NC * NS
ECHUNK = 64
EC = 160
DCHUNK = 128
DC = 80
PC = 16
ROWS_PER_SUB = N_PAD // NS

_mesh = plsc.VectorSubcoreMesh(core_axis_name="c", subcore_axis_name="s")


@functools.partial(
    pl.kernel,
    out_type=jax.ShapeDtypeStruct((NC, 2, N_PAD, 8), jnp.float32),
    mesh=_mesh,
    scratch_types=[
        pltpu.VMEM((DC, 128), jnp.int32),
        pltpu.VMEM((DC, 128), jnp.int32),
        pltpu.VMEM((128, 8), jnp.float32),
        pltpu.SemaphoreType.DMA,
        pltpu.SemaphoreType.DMA,
        pltpu.VMEM_SHARED((N_PAD, 8), jnp.float32),
        pltpu.VMEM_SHARED((N_PAD, 8), jnp.float32),
    ],
)
def _sc_degrees(src_hbm, dst_hbm, zeros_hbm, ones_hbm, out_hbm,
                src_v, dst_v, ones_v, sem_s, sem_d, dsrc_sh, ddst_sh):
    c = lax.axis_index("c")
    s = lax.axis_index("s")
    wid = c * NS + s
    base = s * ROWS_PER_SUB
    pltpu.sync_copy(zeros_hbm, dsrc_sh.at[pl.ds(base, ROWS_PER_SUB)])
    pltpu.sync_copy(zeros_hbm, ddst_sh.at[pl.ds(base, ROWS_PER_SUB)])
    pltpu.sync_copy(ones_hbm, ones_v)
    pltpu.sync_copy(src_hbm.at[wid], src_v)
    pltpu.sync_copy(dst_hbm.at[wid], dst_v)
    plsc.subcore_barrier()

    pltpu.async_copy(ones_v, dsrc_sh.at[src_v.at[0]], sem_s, add=True)
    pltpu.async_copy(ones_v, ddst_sh.at[dst_v.at[0]], sem_d, add=True)

    def body(j, carry):
        pltpu.async_copy(ones_v, dsrc_sh.at[src_v.at[j + 1]], sem_s, add=True)
        pltpu.async_copy(ones_v, ddst_sh.at[dst_v.at[j + 1]], sem_d, add=True)
        pltpu.make_async_copy(ones_v, dsrc_sh.at[src_v.at[j]], sem_s).wait()
        pltpu.make_async_copy(ones_v, ddst_sh.at[dst_v.at[j]], sem_d).wait()
        return carry

    lax.fori_loop(0, DC - 1, body, 0)
    pltpu.make_async_copy(ones_v, dsrc_sh.at[src_v.at[0]], sem_s).wait()
    pltpu.make_async_copy(ones_v, ddst_sh.at[dst_v.at[0]], sem_d).wait()
    plsc.subcore_barrier()
    pltpu.sync_copy(dsrc_sh.at[pl.ds(base, ROWS_PER_SUB)],
                    out_hbm.at[c, 0, pl.ds(base, ROWS_PER_SUB)])
    pltpu.sync_copy(ddst_sh.at[pl.ds(base, ROWS_PER_SUB)],
                    out_hbm.at[c, 1, pl.ds(base, ROWS_PER_SUB)])


NBUF = 4
GRP = EC // NBUF
HALF = D_FEAT // 2


@functools.partial(
    pl.kernel,
    out_type=jax.ShapeDtypeStruct((NC, 2, N_PAD, HALF), jnp.float32),
    mesh=_mesh,
    scratch_types=[
        pltpu.VMEM((EC, ECHUNK), jnp.int32),
        pltpu.VMEM((EC, ECHUNK), jnp.int32),
        pltpu.VMEM((ECHUNK, HALF), jnp.float32),
        pltpu.VMEM((ECHUNK, HALF), jnp.float32),
        pltpu.VMEM((ECHUNK, HALF), jnp.float32),
        pltpu.VMEM((ECHUNK, HALF), jnp.float32),
        pltpu.SemaphoreType.DMA,
        pltpu.SemaphoreType.DMA,
        pltpu.SemaphoreType.DMA,
        pltpu.SemaphoreType.DMA,
        pltpu.VMEM_SHARED((N_PAD, HALF), jnp.float32),
        pltpu.VMEM_SHARED((N_PAD, HALF), jnp.float32),
    ],
    compiler_params=pltpu.CompilerParams(use_tc_tiling_on_sc=False),
)
def _sc_aggregate(h_hbm, src_hbm, dst_hbm, zeros_hbm, out_hbm,
                  src_v, dst_v, r0, r1, r2, r3,
                  sem0, sem1, sem2, sem3,
                  acc_sh, h_sh):
    rows = (r0, r1, r2, r3)
    sems = (sem0, sem1, sem2, sem3)
    c = lax.axis_index("c")
    s = lax.axis_index("s")
    wid = c * NS + s
    base = s * ROWS_PER_SUB
    pltpu.async_copy(src_hbm.at[wid], src_v, sem0).wait()
    pltpu.async_copy(dst_hbm.at[wid], dst_v, sem1).wait()

    def half_body(q, carry):
        pltpu.async_copy(h_hbm.at[q].at[pl.ds(base, ROWS_PER_SUB)],
                         h_sh.at[pl.ds(base, ROWS_PER_SUB)], sem2).wait()
        pltpu.async_copy(zeros_hbm, acc_sh.at[pl.ds(base, ROWS_PER_SUB)],
                         sem0).wait()
        plsc.subcore_barrier()
        for b in range(NBUF):
            pltpu.async_copy(h_sh.at[src_v.at[b]], rows[b], sems[b])

        def group(g, carry2):
            for b in range(NBUF):
                j = g * NBUF + b
                pltpu.make_async_copy(
                    h_sh.at[src_v.at[j]], rows[b], sems[b]).wait()
                pltpu.async_copy(rows[b], acc_sh.at[dst_v.at[j]], sems[b],
                                 add=True)
                pltpu.make_async_copy(
                    rows[b], acc_sh.at[dst_v.at[j]], sems[b]).wait()

                @pl.when(g < GRP - 1)
                def _(j=j, b=b):
                    pltpu.async_copy(
                        h_sh.at[src_v.at[j + NBUF]], rows[b], sems[b])
            return carry2

        lax.fori_loop(0, GRP, group, 0)
        plsc.subcore_barrier()
        pltpu.async_copy(acc_sh.at[pl.ds(base, ROWS_PER_SUB)],
                         out_hbm.at[c, q, pl.ds(base, ROWS_PER_SUB)],
                         sem0).wait()
        plsc.subcore_barrier()
        return carry

    lax.fori_loop(0, 2, half_body, 0)


@functools.partial(
    pl.kernel,
    out_type=[jax.ShapeDtypeStruct((P_EDGES,), jnp.float32)
              for _ in range(2)],
    mesh=_mesh,
    scratch_types=[
        pltpu.VMEM((PC, 128), jnp.int32),
        pltpu.VMEM((PC, 128), jnp.int32),
        pltpu.VMEM((PC, 128), jnp.int32),
        pltpu.VMEM((PC, 128), jnp.int32),
        pltpu.VMEM((128, R_DIM), jnp.float32),
        pltpu.VMEM((128, R_DIM), jnp.float32),
        pltpu.VMEM((128, R_DIM), jnp.float32),
        pltpu.VMEM((128, R_DIM), jnp.float32),
        pltpu.VMEM((128,), jnp.float32),
        pltpu.VMEM((128,), jnp.float32),
        pltpu.SemaphoreType.DMA,
        pltpu.SemaphoreType.DMA,
        pltpu.SemaphoreType.DMA,
        pltpu.SemaphoreType.DMA,
        pltpu.VMEM_SHARED((N_PAD, R_DIM), jnp.float32),
        pltpu.VMEM_SHARED((NUM_RELS, R_DIM), jnp.float32),
    ],
    compiler_params=pltpu.CompilerParams(use_tc_tiling_on_sc=False,
                                         needs_layout_passes=False),
)
def _sc_scores(xn_hbm, rel_hbm, psrc_hbm, pdst_hbm, ndst_hbm, ptype_hbm,
               pos_hbm, neg_hbm,
               ihe, ite, itn, ire, he_v, te_v, tn_v, re_v, pos_c, neg_c,
               sem0, sem1, sem2, sem3, xn_sh, rel_sh):
    c = lax.axis_index("c")
    s = lax.axis_index("s")
    wid = c * NS + s
    base = s * ROWS_PER_SUB
    obase = wid * (PC * 128)
    pltpu.async_copy(xn_hbm.at[pl.ds(base, ROWS_PER_SUB)],
                     xn_sh.at[pl.ds(base, ROWS_PER_SUB)], sem0).wait()

    @pl.when(s == 0)
    def _():
        pltpu.async_copy(rel_hbm, rel_sh, sem1).wait()

    pltpu.async_copy(psrc_hbm.at[wid], ihe, sem0).wait()
    pltpu.async_copy(pdst_hbm.at[wid], ite, sem1).wait()
    pltpu.async_copy(ndst_hbm.at[wid], itn, sem2).wait()
    pltpu.async_copy(ptype_hbm.at[wid], ire, sem3).wait()
    plsc.subcore_barrier()

    def body(j, carry):
        pltpu.async_copy(xn_sh.at[ihe.at[j]], he_v, sem0)
        pltpu.async_copy(xn_sh.at[ite.at[j]], te_v, sem1)
        pltpu.async_copy(xn_sh.at[itn.at[j]], tn_v, sem2)
        pltpu.async_copy(rel_sh.at[ire.at[j]], re_v, sem3)
        pltpu.make_async_copy(xn_sh.at[ihe.at[j]], he_v, sem0).wait()
        pltpu.make_async_copy(xn_sh.at[ite.at[j]], te_v, sem1).wait()
        pltpu.make_async_copy(xn_sh.at[itn.at[j]], tn_v, sem2).wait()
        pltpu.make_async_copy(rel_sh.at[ire.at[j]], re_v, sem3).wait()
        lanes = lax.iota(jnp.int32, 16)
        for e16 in range(8):
            pv = jnp.zeros((16,), jnp.float32)
            nv = jnp.zeros((16,), jnp.float32)
            for l in range(16):
                e = e16 * 16 + l
                hr0 = he_v[e, pl.ds(0, 16)] + re_v[e, pl.ds(0, 16)]
                hr1 = he_v[e, pl.ds(16, 16)] + re_v[e, pl.ds(16, 16)]
                dp0 = hr0 - te_v[e, pl.ds(0, 16)]
                dp1 = hr1 - te_v[e, pl.ds(16, 16)]
                dn0 = hr0 - tn_v[e, pl.ds(0, 16)]
                dn1 = hr1 - tn_v[e, pl.ds(16, 16)]
                pv = jnp.where(lanes == l, jnp.sum(dp0 * dp0 + dp1 * dp1), pv)
                nv = jnp.where(lanes == l, jnp.sum(dn0 * dn0 + dn1 * dn1), nv)
            pos_c[pl.ds(e16 * 16, 16)] = pv
            neg_c[pl.ds(e16 * 16, 16)] = nv
        pltpu.async_copy(pos_c, pos_hbm.at[pl.ds(obase + j * 128, 128)],
                         sem0).wait()
        pltpu.async_copy(neg_c, neg_hbm.at[pl.ds(obase + j * 128, 128)],
                         sem1).wait()
        return carry

    lax.fori_loop(0, PC, body, 0)


def _tc_norms_body(dp_ref, feat_ref, h1_ref, nsrc_ref, ndst_ref):
    dsrc = dp_ref[0, 0] + dp_ref[1, 0]
    ddst = dp_ref[0, 1] + dp_ref[1, 1]
    dsrc1 = dsrc[:, 0:1]
    ddst1 = ddst[:, 0:1]
    nsrc = jnp.where(dsrc1 > 0.0, lax.rsqrt(dsrc1), 0.0)
    ndst = jnp.where(ddst1 > 0.0, lax.rsqrt(ddst1), 0.0)
    nsrc_ref[...] = jnp.broadcast_to(nsrc, (N_PAD, D_FEAT))
    ndst_ref[...] = jnp.broadcast_to(ndst, (N_PAD, D_FEAT))
    h1 = feat_ref[...] * nsrc
    h1_ref[0] = h1[:, :HALF]
    h1_ref[1] = h1[:, HALF:]


def _sum_parts(p_ref, ndst_ref):
    lo = p_ref[0, 0] + p_ref[1, 0]
    hi = p_ref[0, 1] + p_ref[1, 1]
    return jnp.concatenate([lo, hi], axis=1) * ndst_ref[...]


def _tc_layer1_body(p_ref, ndst_ref, nsrc_ref, w_ref, b_ref, h2_ref):
    agg = _sum_parts(p_ref, ndst_ref)
    x1 = jnp.tanh(jnp.dot(agg, w_ref[...],
                          preferred_element_type=jnp.float32) + b_ref[...])
    h2 = x1 * nsrc_ref[...]
    h2_ref[0] = h2[:, :HALF]
    h2_ref[1] = h2[:, HALF:]


def _tc_layer2_body(p_ref, ndst_ref, feat_ref, w_ref, b_ref, wl_ref, rel_ref,
                    xn_ref, reln_ref):
    agg = _sum_parts(p_ref, ndst_ref)
    x2 = jnp.tanh(jnp.dot(agg, w_ref[...],
                          preferred_element_type=jnp.float32) + b_ref[...])
    x = (jnp.dot(x2, wl_ref[0:D_FEAT], preferred_element_type=jnp.float32)
         + jnp.dot(feat_ref[...], wl_ref[D_FEAT:],
                   preferred_element_type=jnp.float32))
    n = jnp.sqrt(jnp.sum(x * x, axis=1, keepdims=True))
    xn_ref[...] = x / jnp.maximum(n, 1e-12)
    r = rel_ref[...]
    rn = jnp.sqrt(jnp.sum(r * r, axis=1, keepdims=True))
    reln_ref[...] = r / jnp.maximum(rn, 1.0)


def _tc_sqrt_body(p_ref, n_ref, pos_ref, neg_ref):
    pos_ref[...] = jnp.sqrt(p_ref[...])
    neg_ref[...] = jnp.sqrt(n_ref[...])


def kernel(input_feat, edge_src, edge_dst, pos_src, pos_dst, pos_type, neg_dst,
           W1, b1, W2, b2, W_lin, rel_table):
    f32 = jnp.float32
    feat_p = jnp.pad(input_feat, ((0, N_PAD - N_NODES), (0, 0)))
    epad = E_PAD - E_EDGES
    src_flat = jnp.concatenate(
        [edge_src, jnp.full((epad,), N_NODES, jnp.int32)])
    dst_flat = jnp.concatenate(
        [edge_dst, jnp.full((epad,), N_NODES, jnp.int32)])
    src_d = src_flat.reshape(NW, DC, DCHUNK)
    dst_d = dst_flat.reshape(NW, DC, DCHUNK)
    src_p = src_flat.reshape(NW, EC, ECHUNK)
    dst_p = dst_flat.reshape(NW, EC, ECHUNK)
    psrc_r = pos_src.reshape(NW, PC, 128)
    pdst_r = pos_dst.reshape(NW, PC, 128)
    ndst_r = neg_dst.reshape(NW, PC, 128)
    ptype_r = pos_type.reshape(NW, PC, 128)
    zeros_blk = jnp.zeros((ROWS_PER_SUB, HALF), f32)
    zeros_blk8 = jnp.zeros((ROWS_PER_SUB, 8), f32)
    ones_blk = jnp.ones((128, 8), f32)

    deg_part = _sc_degrees(src_d, dst_d, zeros_blk8, ones_blk)
    h1, nsrc_b, ndst_b = pl.pallas_call(
        _tc_norms_body,
        out_shape=[jax.ShapeDtypeStruct((2, N_PAD, HALF), f32),
                   jax.ShapeDtypeStruct((N_PAD, D_FEAT), f32),
                   jax.ShapeDtypeStruct((N_PAD, D_FEAT), f32)],
    )(deg_part, feat_p)

    p1 = _sc_aggregate(h1, src_p, dst_p, zeros_blk)
    h2 = pl.pallas_call(
        _tc_layer1_body,
        out_shape=jax.ShapeDtypeStruct((2, N_PAD, HALF), f32),
    )(p1, ndst_b, nsrc_b, W1, b1)

    p2 = _sc_aggregate(h2, src_p, dst_p, zeros_blk)
    xn, rel_n = pl.pallas_call(
        _tc_layer2_body,
        out_shape=[jax.ShapeDtypeStruct((N_PAD, R_DIM), f32),
                   jax.ShapeDtypeStruct((NUM_RELS, R_DIM), f32)],
    )(p2, ndst_b, feat_p, W2, b2, W_lin, rel_table)

    pos_sq, neg_sq = _sc_scores(xn, rel_n, psrc_r, pdst_r, ndst_r, ptype_r)
    pos, neg = pl.pallas_call(
        _tc_sqrt_body,
        out_shape=[jax.ShapeDtypeStruct((P_EDGES,), f32)] * 2,
    )(pos_sq, neg_sq)
    return (pos, neg)
